# baseline (speedup 1.0000x reference)
import functools

import jax
import jax.numpy as jnp
import numpy as np
from jax.experimental import pallas as pl
from jax.experimental.pallas import tpu as pltpu

N_BINS = 15
NB_PAD = 16


def _bin_bounds(num_samples):
    bi = jax.lax.broadcasted_iota(
        jnp.int32, (1, NB_PAD), 1).astype(jnp.float32)
    scale = np.float32(num_samples) / np.float32(N_BINS)
    return jnp.round(bi * scale), jnp.round((bi + 1.0) * scale)


def _partials_kernel(logits_ref, labels_ref, out_ref, *, num_samples, n_cols):
    x = logits_ref[...]
    m = jnp.max(x, axis=1, keepdims=True)
    s = jnp.sum(jnp.exp(x - m), axis=1, keepdims=True)
    conf = 1.0 / s
    col = jax.lax.broadcasted_iota(jnp.int32, x.shape, 1)
    pred = jnp.min(jnp.where(x == m, col, n_cols), axis=1, keepdims=True)
    acc = (pred == labels_ref[...]).astype(jnp.float32)

    lo, up = _bin_bounds(num_samples)
    in_bin = ((conf > lo) & (conf <= up)).astype(jnp.float32)
    cnt = jnp.sum(in_bin, axis=0, keepdims=True)
    sconf = jnp.sum(in_bin * conf, axis=0, keepdims=True)
    sacc = jnp.sum(in_bin * acc, axis=0, keepdims=True)
    out_ref[0] = jnp.concatenate([cnt, sconf, sacc], axis=0)


def _finish_kernel(part_ref, out_ref, *, num_samples):
    tot = jnp.sum(part_ref[...], axis=0)
    cnt_f = tot[0:1, :]
    denom = jnp.maximum(cnt_f, 1.0)
    avg_conf = tot[1:2, :] / denom
    avg_acc = tot[2:3, :] / denom
    prop = cnt_f / np.float32(num_samples)
    out_ref[0] = jnp.sum(jnp.abs(avg_conf - avg_acc) * prop)


@jax.jit
def kernel(logits, labels):
    num_samples, n_cols = logits.shape
    blk = 512
    n_steps = num_samples // blk
    labels2d = labels.astype(jnp.int32).reshape(num_samples, 1)

    partials = pl.pallas_call(
        functools.partial(_partials_kernel, num_samples=num_samples,
                          n_cols=n_cols),
        grid=(n_steps,),
        in_specs=[
            pl.BlockSpec((blk, n_cols), lambda i: (i, 0)),
            pl.BlockSpec((blk, 1), lambda i: (i, 0)),
        ],
        out_specs=pl.BlockSpec((1, 3, NB_PAD), lambda i: (i, 0, 0)),
        out_shape=jax.ShapeDtypeStruct((n_steps, 3, NB_PAD), jnp.float32),
        compiler_params=pltpu.CompilerParams(
            dimension_semantics=("parallel",),
        ),
    )(logits, labels2d)

    ece = pl.pallas_call(
        functools.partial(_finish_kernel, num_samples=num_samples),
        out_specs=pl.BlockSpec(memory_space=pltpu.SMEM),
        out_shape=jax.ShapeDtypeStruct((1,), jnp.float32),
    )(partials)
    return ece
